# baseline (speedup 1.0000x reference)
import functools

import numpy as np
import jax
import jax.numpy as jnp
from jax import lax
from jax.experimental import pallas as pl
from jax.experimental.pallas import tpu as pltpu
from jax.experimental.pallas import tpu_sc as plsc

N = 10000
NP = 10240
E = 320000
D = 128
NUM_LAYERS = 4
ALPHA = 0.1
THETA = 0.5

NC = 2
NS = 16
NW = NC * NS
CH = 64
EP = 327680
EW = EP // NW
NCHUNK = EW // CH
CPS = 16
SEG = NCHUNK // CPS
NBUF = 4
RPT = NP // NS

_mesh = plsc.VectorSubcoreMesh(core_axis_name="c", subcore_axis_name="s")


def _fill(buf, val):
    v16 = jnp.full((16,), val, jnp.float32)

    def row(i, carry):
        for j in range(D // 16):
            buf[i, pl.ds(j * 16, 16)] = v16
        return carry

    lax.fori_loop(0, CH, row, 0)


def _zero_acc(acc_sh, zb_v, s):
    for k in range(RPT // CH):
        pltpu.sync_copy(zb_v, acc_sh.at[pl.ds(s * RPT + k * CH, CH)])


@functools.partial(
    pl.kernel,
    mesh=_mesh,
    out_type=jax.ShapeDtypeStruct((NC, NP, D), jnp.float32),
    scratch_types=[
        pltpu.VMEM_SHARED((NP, D), jnp.float32),
        pltpu.VMEM((CH, D), jnp.float32),
        pltpu.VMEM((CPS, CH), jnp.int32),
        pltpu.SemaphoreType.DMA,
    ],
)
def _deg_kernel(dstr_hbm, out_hbm, acc_sh, ones_v, didx_v, sem):
    c = lax.axis_index("c")
    s = lax.axis_index("s")
    wid = c * NS + s

    _fill(ones_v, 0.0)
    _zero_acc(acc_sh, ones_v, s)
    _fill(ones_v, 1.0)
    plsc.subcore_barrier()

    for seg in range(SEG):
        pltpu.sync_copy(
            dstr_hbm.at[pl.ds(wid * NCHUNK + seg * CPS, CPS)], didx_v)

        def wave(p, carry):
            for b in range(4):
                pltpu.async_copy(ones_v, acc_sh.at[didx_v.at[p * 4 + b]],
                                 sem, add=True)
            for b in range(4):
                pltpu.make_async_copy(ones_v, acc_sh.at[didx_v.at[0]],
                                      sem).wait()
            return carry

        lax.fori_loop(0, CPS // 4, wave, 0)
    plsc.subcore_barrier()
    pltpu.sync_copy(acc_sh.at[pl.ds(s * RPT, RPT)],
                    out_hbm.at[c, pl.ds(s * RPT, RPT)])


@functools.partial(
    pl.kernel,
    mesh=_mesh,
    out_type=jax.ShapeDtypeStruct((NC, NP, D), jnp.float32),
    scratch_types=[
        pltpu.VMEM_SHARED((NP, D), jnp.float32),
        pltpu.VMEM((CPS, CH), jnp.int32),
        pltpu.VMEM((CPS, CH), jnp.int32),
    ]
    + [pltpu.VMEM((CH, D), jnp.float32)] * NBUF
    + [pltpu.SemaphoreType.DMA] * (2 * NBUF),
)
def _prop_kernel(hp_hbm, srcr_hbm, dstr_hbm, out_hbm,
                 acc_sh, sidx_v, didx_v, *ring):
    rows = ring[:NBUF]
    gsem = ring[NBUF:2 * NBUF]
    ssem = ring[2 * NBUF:]
    c = lax.axis_index("c")
    s = lax.axis_index("s")
    wid = c * NS + s

    _fill(rows[0], 0.0)
    _zero_acc(acc_sh, rows[0], s)
    plsc.subcore_barrier()

    for seg in range(SEG):
        pltpu.sync_copy(
            srcr_hbm.at[pl.ds(wid * NCHUNK + seg * CPS, CPS)], sidx_v)
        pltpu.sync_copy(
            dstr_hbm.at[pl.ds(wid * NCHUNK + seg * CPS, CPS)], didx_v)
        gathers = [
            pltpu.async_copy(hp_hbm.at[sidx_v.at[b]], rows[b], gsem[b])
            for b in range(NBUF)
        ]
        for b in range(NBUF):
            gathers[b].wait()
            pltpu.async_copy(rows[b], acc_sh.at[didx_v.at[b]],
                             ssem[b], add=True)

        def steady(p, carry):
            for b in range(NBUF):
                ci = p * NBUF + b
                pltpu.make_async_copy(rows[b], acc_sh.at[didx_v.at[0]],
                                      ssem[b]).wait()
                cp = pltpu.async_copy(hp_hbm.at[sidx_v.at[ci]], rows[b],
                                      gsem[b])
                cp.wait()
                pltpu.async_copy(rows[b], acc_sh.at[didx_v.at[ci]],
                                 ssem[b], add=True)
            return carry

        lax.fori_loop(1, CPS // NBUF, steady, 0)
        for b in range(NBUF):
            pltpu.make_async_copy(rows[b], acc_sh.at[didx_v.at[0]],
                                  ssem[b]).wait()
    plsc.subcore_barrier()
    pltpu.sync_copy(acc_sh.at[pl.ds(s * RPT, RPT)],
                    out_hbm.at[c, pl.ds(s * RPT, RPT)])


BR = 1024
GN = NP // BR


def _init_body(x_ref, w_ref, b_ref, p_ref, x0_ref, hp_ref, dinv_ref):
    deg = 1.0 + p_ref[0, :, 0:1] + p_ref[1, :, 0:1]
    dinv = jnp.broadcast_to(lax.rsqrt(deg), (BR, D))
    x0 = jnp.dot(x_ref[...], w_ref[...],
                 preferred_element_type=jnp.float32) + b_ref[...]
    x0_ref[...] = x0
    hp_ref[...] = dinv * x0
    dinv_ref[...] = dinv


_init_call = pl.pallas_call(
    _init_body,
    grid=(GN,),
    in_specs=[
        pl.BlockSpec((BR, D), lambda i: (i, 0)),
        pl.BlockSpec((D, D), lambda i: (0, 0)),
        pl.BlockSpec((1, D), lambda i: (0, 0)),
        pl.BlockSpec((NC, BR, D), lambda i: (0, i, 0)),
    ],
    out_specs=[pl.BlockSpec((BR, D), lambda i: (i, 0))] * 3,
    out_shape=[jax.ShapeDtypeStruct((NP, D), jnp.float32)] * 3,
)


def _layer_body(p_ref, h_ref, hp_ref, x0_ref, dinv_ref, w_ref,
                hn_ref, hpn_ref, *, beta, last):
    dv = dinv_ref[...]
    agg = dv * (p_ref[0] + p_ref[1] + hp_ref[...])
    out = (1.0 - ALPHA) * agg + ALPHA * x0_ref[...]
    raw = (1.0 - beta) * out + beta * jnp.dot(
        out, w_ref[...], preferred_element_type=jnp.float32)
    hn = h_ref[...] + raw
    if not last:
        hn = jnp.maximum(hn, 0.0)
    hn_ref[...] = hn
    hpn_ref[...] = dv * hn


def _make_layer_call(beta, last):
    return pl.pallas_call(
        functools.partial(_layer_body, beta=beta, last=last),
        grid=(GN,),
        in_specs=[
            pl.BlockSpec((NC, BR, D), lambda i: (0, i, 0)),
            pl.BlockSpec((BR, D), lambda i: (i, 0)),
            pl.BlockSpec((BR, D), lambda i: (i, 0)),
            pl.BlockSpec((BR, D), lambda i: (i, 0)),
            pl.BlockSpec((BR, D), lambda i: (i, 0)),
            pl.BlockSpec((D, D), lambda i: (0, 0)),
        ],
        out_specs=[pl.BlockSpec((BR, D), lambda i: (i, 0))] * 2,
        out_shape=[jax.ShapeDtypeStruct((NP, D), jnp.float32)] * 2,
    )


_layer_calls = [
    _make_layer_call(float(np.log(THETA / (i + 1) + 1.0)), i == NUM_LAYERS - 1)
    for i in range(NUM_LAYERS)
]


def kernel(x, edge_index, W_proj, b_proj, Ws):
    npad = EP - E
    pad_src = (jnp.arange(npad, dtype=jnp.int32) * 37) % N
    pad_dst = N + (jnp.arange(npad, dtype=jnp.int32) % (NP - N))
    srcr = jnp.concatenate([edge_index[0], pad_src]).reshape(NW * NCHUNK, CH)
    dstr = jnp.concatenate([edge_index[1], pad_dst]).reshape(NW * NCHUNK, CH)
    xp = jnp.pad(x, ((0, NP - N), (0, 0)))
    degp = _deg_kernel(dstr)
    x0, hp, dinv = _init_call(xp, W_proj, b_proj.reshape(1, D), degp)
    h = x0
    for i in range(NUM_LAYERS):
        part = _prop_kernel(hp, srcr, dstr)
        h, hp = _layer_calls[i](part, h, hp, x0, dinv, Ws[i])
    return h[:N]


# speedup vs baseline: 1.2951x; 1.2951x over previous
import functools

import numpy as np
import jax
import jax.numpy as jnp
from jax import lax
from jax.experimental import pallas as pl
from jax.experimental.pallas import tpu as pltpu
from jax.experimental.pallas import tpu_sc as plsc

N = 10000
NP = 10240
E = 320000
D = 128
NUM_LAYERS = 4
ALPHA = 0.1
THETA = 0.5

NC = 2
NS = 16
NW = NC * NS
CH = 128
EP = 327680
EW = EP // NW
NCHUNK = EW // CH
CPS = 40
SEG = NCHUNK // CPS
NBUF = 2
RPT = NP // NS

_mesh = plsc.VectorSubcoreMesh(core_axis_name="c", subcore_axis_name="s")


def _fill(buf, val):
    v16 = jnp.full((16,), val, jnp.float32)

    def row(i, carry):
        for j in range(D // 16):
            buf[i, pl.ds(j * 16, 16)] = v16
        return carry

    lax.fori_loop(0, CH, row, 0)


def _zero_acc(acc_sh, zb_v, s):
    for k in range(RPT // CH):
        pltpu.sync_copy(zb_v, acc_sh.at[pl.ds(s * RPT + k * CH, CH)])


@functools.partial(
    pl.kernel,
    mesh=_mesh,
    out_type=jax.ShapeDtypeStruct((NC, NP, D), jnp.float32),
    scratch_types=[
        pltpu.VMEM_SHARED((NP, D), jnp.float32),
        pltpu.VMEM((CH, D), jnp.float32),
        pltpu.VMEM((CPS, CH), jnp.int32),
        pltpu.SemaphoreType.DMA,
    ],
)
def _deg_kernel(dstr_hbm, out_hbm, acc_sh, ones_v, didx_v, sem):
    c = lax.axis_index("c")
    s = lax.axis_index("s")
    wid = c * NS + s

    _fill(ones_v, 0.0)
    _zero_acc(acc_sh, ones_v, s)
    _fill(ones_v, 1.0)
    plsc.subcore_barrier()

    for seg in range(SEG):
        pltpu.sync_copy(
            dstr_hbm.at[pl.ds(wid * NCHUNK + seg * CPS, CPS)], didx_v)

        def wave(p, carry):
            for b in range(4):
                pltpu.async_copy(ones_v, acc_sh.at[didx_v.at[p * 4 + b]],
                                 sem, add=True)
            for b in range(4):
                pltpu.make_async_copy(ones_v, acc_sh.at[didx_v.at[0]],
                                      sem).wait()
            return carry

        lax.fori_loop(0, CPS // 4, wave, 0)
    plsc.subcore_barrier()
    pltpu.sync_copy(acc_sh.at[pl.ds(s * RPT, RPT)],
                    out_hbm.at[c, pl.ds(s * RPT, RPT)])


@functools.partial(
    pl.kernel,
    mesh=_mesh,
    out_type=jax.ShapeDtypeStruct((NC, NP, D), jnp.float32),
    scratch_types=[
        pltpu.VMEM_SHARED((NP, D), jnp.float32),
        pltpu.VMEM((CPS, CH), jnp.int32),
        pltpu.VMEM((CPS, CH), jnp.int32),
    ]
    + [pltpu.VMEM((CH, D), jnp.float32)] * NBUF
    + [pltpu.SemaphoreType.DMA] * (2 * NBUF),
)
def _prop_kernel(hp_hbm, srcr_hbm, dstr_hbm, out_hbm,
                 acc_sh, sidx_v, didx_v, *ring):
    rows = ring[:NBUF]
    gsem = ring[NBUF:2 * NBUF]
    ssem = ring[2 * NBUF:]
    c = lax.axis_index("c")
    s = lax.axis_index("s")
    wid = c * NS + s

    _fill(rows[0], 0.0)
    _zero_acc(acc_sh, rows[0], s)
    plsc.subcore_barrier()

    for seg in range(SEG):
        pltpu.sync_copy(
            srcr_hbm.at[pl.ds(wid * NCHUNK + seg * CPS, CPS)], sidx_v)
        pltpu.sync_copy(
            dstr_hbm.at[pl.ds(wid * NCHUNK + seg * CPS, CPS)], didx_v)
        gathers = [
            pltpu.async_copy(hp_hbm.at[sidx_v.at[b]], rows[b], gsem[b])
            for b in range(NBUF)
        ]
        for b in range(NBUF):
            gathers[b].wait()
            pltpu.async_copy(rows[b], acc_sh.at[didx_v.at[b]],
                             ssem[b], add=True)

        def steady(p, carry):
            for b in range(NBUF):
                ci = p * NBUF + b
                pltpu.make_async_copy(rows[b], acc_sh.at[didx_v.at[0]],
                                      ssem[b]).wait()
                cp = pltpu.async_copy(hp_hbm.at[sidx_v.at[ci]], rows[b],
                                      gsem[b])
                cp.wait()
                pltpu.async_copy(rows[b], acc_sh.at[didx_v.at[ci]],
                                 ssem[b], add=True)
            return carry

        lax.fori_loop(1, CPS // NBUF, steady, 0)
        for b in range(NBUF):
            pltpu.make_async_copy(rows[b], acc_sh.at[didx_v.at[0]],
                                  ssem[b]).wait()
    plsc.subcore_barrier()
    pltpu.sync_copy(acc_sh.at[pl.ds(s * RPT, RPT)],
                    out_hbm.at[c, pl.ds(s * RPT, RPT)])


BR = 1024
GN = NP // BR


def _init_body(x_ref, w_ref, b_ref, p_ref, x0_ref, hp_ref, dinv_ref):
    deg = 1.0 + p_ref[0, :, 0:1] + p_ref[1, :, 0:1]
    dinv = jnp.broadcast_to(lax.rsqrt(deg), (BR, D))
    x0 = jnp.dot(x_ref[...], w_ref[...],
                 preferred_element_type=jnp.float32) + b_ref[...]
    x0_ref[...] = x0
    hp_ref[...] = dinv * x0
    dinv_ref[...] = dinv


_init_call = pl.pallas_call(
    _init_body,
    grid=(GN,),
    in_specs=[
        pl.BlockSpec((BR, D), lambda i: (i, 0)),
        pl.BlockSpec((D, D), lambda i: (0, 0)),
        pl.BlockSpec((1, D), lambda i: (0, 0)),
        pl.BlockSpec((NC, BR, D), lambda i: (0, i, 0)),
    ],
    out_specs=[pl.BlockSpec((BR, D), lambda i: (i, 0))] * 3,
    out_shape=[jax.ShapeDtypeStruct((NP, D), jnp.float32)] * 3,
)


def _layer_body(p_ref, h_ref, hp_ref, x0_ref, dinv_ref, w_ref,
                hn_ref, hpn_ref, *, beta, last):
    dv = dinv_ref[...]
    agg = dv * (p_ref[0] + p_ref[1] + hp_ref[...])
    out = (1.0 - ALPHA) * agg + ALPHA * x0_ref[...]
    raw = (1.0 - beta) * out + beta * jnp.dot(
        out, w_ref[...], preferred_element_type=jnp.float32)
    hn = h_ref[...] + raw
    if not last:
        hn = jnp.maximum(hn, 0.0)
    hn_ref[...] = hn
    hpn_ref[...] = dv * hn


def _make_layer_call(beta, last):
    return pl.pallas_call(
        functools.partial(_layer_body, beta=beta, last=last),
        grid=(GN,),
        in_specs=[
            pl.BlockSpec((NC, BR, D), lambda i: (0, i, 0)),
            pl.BlockSpec((BR, D), lambda i: (i, 0)),
            pl.BlockSpec((BR, D), lambda i: (i, 0)),
            pl.BlockSpec((BR, D), lambda i: (i, 0)),
            pl.BlockSpec((BR, D), lambda i: (i, 0)),
            pl.BlockSpec((D, D), lambda i: (0, 0)),
        ],
        out_specs=[pl.BlockSpec((BR, D), lambda i: (i, 0))] * 2,
        out_shape=[jax.ShapeDtypeStruct((NP, D), jnp.float32)] * 2,
    )


_layer_calls = [
    _make_layer_call(float(np.log(THETA / (i + 1) + 1.0)), i == NUM_LAYERS - 1)
    for i in range(NUM_LAYERS)
]


def kernel(x, edge_index, W_proj, b_proj, Ws):
    npad = EP - E
    pad_src = (jnp.arange(npad, dtype=jnp.int32) * 37) % N
    pad_dst = N + (jnp.arange(npad, dtype=jnp.int32) % (NP - N))
    srcr = jnp.concatenate([edge_index[0], pad_src]).reshape(NW * NCHUNK, CH)
    dstr = jnp.concatenate([edge_index[1], pad_dst]).reshape(NW * NCHUNK, CH)
    xp = jnp.pad(x, ((0, NP - N), (0, 0)))
    degp = _deg_kernel(dstr)
    x0, hp, dinv = _init_call(xp, W_proj, b_proj.reshape(1, D), degp)
    h = x0
    for i in range(NUM_LAYERS):
        part = _prop_kernel(hp, srcr, dstr)
        h, hp = _layer_calls[i](part, h, hp, x0, dinv, Ws[i])
    return h[:N]
